# baseline (speedup 1.0000x reference)
import jax
import jax.numpy as jnp
from jax import lax
from jax.experimental import pallas as pl
from jax.experimental.pallas import tpu as pltpu
from jax.experimental.pallas import tpu_sc as plsc

N_NODES = 10000
N_EDGES = 320000
D = 128
DW = 144
NC, NS = 2, 16
NW = NC * NS
NPAD = 10240
NHALF = NPAD // 2
DUMPR = 128
ACCR = NHALF + DUMPR
RPT1 = ACCR // NS

EPT1 = N_EDGES // NS
C1 = 400
NCHUNK1 = EPT1 // C1

EPW = N_EDGES // NW
C3 = 80
NCHUNK3 = EPW // C3
CP = 80
NCHUNKP = EPW // CP

_mesh = plsc.VectorSubcoreMesh(
    core_axis_name="c", subcore_axis_name="s", num_cores=NC, num_subcores=NS
)
_sc_params = pltpu.CompilerParams(
    needs_layout_passes=False, use_tc_tiling_on_sc=False
)


def _sums_body(edge_hbm, idx_hbm, z_hbm, onescol_hbm, out_hbm,
               ebuf, ibuf, wbuf, acc):
    cid = lax.axis_index("c")
    sid = lax.axis_index("s")
    r0 = sid * RPT1
    pltpu.sync_copy(z_hbm.at[pl.ds(r0, RPT1)], acc.at[pl.ds(r0, RPT1)])

    lanes = lax.iota(jnp.int32, 16)
    pltpu.sync_copy(onescol_hbm, ebuf.at[pl.ds(0, C1), pl.ds(D, 16)])
    plsc.subcore_barrier()

    lo = cid * NHALF
    base0 = sid * EPT1

    def chunk(c, carry):
        base = base0 + c * C1
        pltpu.sync_copy(idx_hbm.at[pl.ds(base, C1)], ibuf)
        pltpu.sync_copy(edge_hbm.at[pl.ds(base, C1)],
                        ebuf.at[pl.ds(0, C1), pl.ds(0, D)])

        def grp(j, cc):
            iv = ibuf[pl.ds(j * 16, 16)]
            local = iv - lo
            ok = (local >= 0) & (local < NHALF)
            wbuf[pl.ds(j * 16, 16)] = jnp.where(ok, local, NHALF + lanes)
            return cc

        lax.fori_loop(0, C1 // 16, grp, 0)
        pltpu.sync_copy(ebuf, acc.at[wbuf], add=True)
        return carry

    lax.fori_loop(0, NCHUNK1, chunk, 0)
    plsc.subcore_barrier()
    pltpu.sync_copy(acc.at[pl.ds(r0, RPT1)],
                    out_hbm.at[cid, pl.ds(r0, RPT1)])


_sums_call = pl.kernel(
    _sums_body,
    out_type=jax.ShapeDtypeStruct((NC, ACCR, DW), jnp.float32),
    mesh=_mesh,
    scratch_types=(
        pltpu.VMEM((C1, DW), jnp.float32),
        pltpu.VMEM((C1,), jnp.int32),
        pltpu.VMEM((C1,), jnp.int32),
        pltpu.VMEM_SHARED((ACCR, DW), jnp.float32),
    ),
    compiler_params=_sc_params,
)


BN = 256


def _combine_body(s_ref, na_ref, w0a_ref, w0b_ref, b0_ref,
                  wuv_ref, buv_ref, h_ref, uv_ref):
    s = s_ref[...]
    cnt = s[:, D:D + 1]
    mean = s[:, 0:D] / jnp.maximum(cnt, 1.0)
    acc = jnp.dot(na_ref[...], w0a_ref[...], preferred_element_type=jnp.float32)
    acc = acc + jnp.dot(mean, w0b_ref[...], preferred_element_type=jnp.float32)
    h = jax.nn.sigmoid(acc + b0_ref[...])
    h_ref[...] = h
    uv_ref[...] = jnp.dot(h, wuv_ref[...], preferred_element_type=jnp.float32) + buv_ref[...]


_combine_call = pl.pallas_call(
    _combine_body,
    grid=(NPAD // BN,),
    in_specs=[
        pl.BlockSpec((BN, DW), lambda i: (i, 0)),
        pl.BlockSpec((BN, D), lambda i: (i, 0)),
        pl.BlockSpec((D, D), lambda i: (0, 0)),
        pl.BlockSpec((D, D), lambda i: (0, 0)),
        pl.BlockSpec((1, D), lambda i: (0, 0)),
        pl.BlockSpec((D, 16), lambda i: (0, 0)),
        pl.BlockSpec((1, 16), lambda i: (0, 0)),
    ],
    out_specs=[
        pl.BlockSpec((BN, D), lambda i: (i, 0)),
        pl.BlockSpec((BN, 16), lambda i: (i, 0)),
    ],
    out_shape=[
        jax.ShapeDtypeStruct((NPAD, D), jnp.float32),
        jax.ShapeDtypeStruct((NPAD, 16), jnp.float32),
    ],
)


def _edge_body(hp_hbm, src_hbm, dst_hbm, emb_hbm, probs_hbm,
               sbuf, dbuf, rowss, rowsd, pbuf, sem):
    cid = lax.axis_index("c")
    sid = lax.axis_index("s")
    wid = cid * NS + sid
    base0 = wid * EPW
    lanes = lax.iota(jnp.int32, 16)
    cu = jnp.full((16,), D, jnp.int32)
    cv = jnp.full((16,), D + 1, jnp.int32)

    def chunk(c, carry):
        base = base0 + c * C3
        pltpu.sync_copy(src_hbm.at[pl.ds(base, C3)], sbuf)
        pltpu.sync_copy(dst_hbm.at[pl.ds(base, C3)], dbuf)
        cps = pltpu.async_copy(hp_hbm.at[sbuf], rowss, sem)
        cpd = pltpu.async_copy(hp_hbm.at[dbuf], rowsd, sem)
        cps.wait()
        cpd.wait()
        pltpu.sync_copy(rowss.at[pl.ds(0, C3), pl.ds(0, D)],
                        emb_hbm.at[pl.ds(base, C3), pl.ds(0, D)])
        pltpu.sync_copy(rowsd.at[pl.ds(0, C3), pl.ds(0, D)],
                        emb_hbm.at[pl.ds(base, C3), pl.ds(D, D)])

        def grp(j, cc):
            r = j * 16 + lanes
            u = plsc.load_gather(rowss, [r, cu])
            v = plsc.load_gather(rowsd, [r, cv])
            x = u + v
            pbuf[pl.ds(j * 16, 16)] = 1.0 / (1.0 + jnp.exp(-x))
            return cc

        lax.fori_loop(0, C3 // 16, grp, 0)
        pltpu.sync_copy(pbuf, probs_hbm.at[pl.ds(base, C3)])
        return carry

    lax.fori_loop(0, NCHUNK3, chunk, 0)


_edge_call = pl.kernel(
    _edge_body,
    out_type=(
        jax.ShapeDtypeStruct((N_EDGES, 2 * D), jnp.float32),
        jax.ShapeDtypeStruct((N_EDGES,), jnp.float32),
    ),
    mesh=_mesh,
    scratch_types=(
        pltpu.VMEM((C3,), jnp.int32),
        pltpu.VMEM((C3,), jnp.int32),
        pltpu.VMEM((C3, DW), jnp.float32),
        pltpu.VMEM((C3, DW), jnp.float32),
        pltpu.VMEM((C3,), jnp.float32),
        pltpu.SemaphoreType.DMA,
    ),
    compiler_params=_sc_params,
)


def kernel(edge_attr, edge_index, node_attr, W0, b0, W_out, b_out):
    src = edge_index[0].astype(jnp.int32)
    dst = edge_index[1].astype(jnp.int32)

    z = jnp.zeros((ACCR, DW), jnp.float32)
    onescol = jnp.zeros((C1, 16), jnp.float32).at[:, 0].set(1.0)
    ps = _sums_call(edge_attr, src, z, onescol)
    sums2 = jnp.concatenate([ps[0, :NHALF], ps[1, :NHALF]], axis=0)

    na_pad = jnp.pad(node_attr, ((0, NPAD - N_NODES), (0, 0)))
    w0a = W0[:, :D].T
    w0b = W0[:, D:].T
    wu = W_out[0, :D]
    wv = W_out[0, D:]
    wuv = jnp.concatenate(
        [jnp.stack([wu, wv], axis=1), jnp.zeros((D, 14), jnp.float32)], axis=1
    )
    buv = jnp.zeros((1, 16), jnp.float32).at[0, 0].set(b_out[0])
    h, uvt = _combine_call(sums2, na_pad, w0a, w0b, b0.reshape(1, D), wuv, buv)
    hp = jnp.concatenate([h, uvt], axis=1)

    emb, probs_flat = _edge_call(hp, src, dst)
    return probs_flat.reshape(N_EDGES, 1), emb


# speedup vs baseline: 1.1626x; 1.1626x over previous
import jax
import jax.numpy as jnp
from jax import lax
from jax.experimental import pallas as pl
from jax.experimental.pallas import tpu as pltpu
from jax.experimental.pallas import tpu_sc as plsc

N_NODES = 10000
N_EDGES = 320000
D = 128
DW = 144
NC, NS = 2, 16
NW = NC * NS
NPAD = 10240
NHALF = NPAD // 2
DUMPR = 128
ACCR = NHALF + DUMPR
RPT1 = ACCR // NS

EPT1 = N_EDGES // NS
C1 = 400
NCHUNK1 = EPT1 // C1

EPW = N_EDGES // NW
C3 = 80
NCHUNK3 = EPW // C3
CP = 80
NCHUNKP = EPW // CP

_mesh = plsc.VectorSubcoreMesh(
    core_axis_name="c", subcore_axis_name="s", num_cores=NC, num_subcores=NS
)
_sc_params = pltpu.CompilerParams(
    needs_layout_passes=False, use_tc_tiling_on_sc=False
)


def _sums_body(edge_hbm, idx_hbm, z_hbm, onescol_hbm, out_hbm,
               ebuf, ibuf, wbuf, acc):
    cid = lax.axis_index("c")
    sid = lax.axis_index("s")
    r0 = sid * RPT1
    pltpu.sync_copy(z_hbm.at[pl.ds(r0, RPT1)], acc.at[pl.ds(r0, RPT1)])

    lanes = lax.iota(jnp.int32, 16)
    pltpu.sync_copy(onescol_hbm, ebuf.at[pl.ds(0, C1), pl.ds(D, 16)])
    plsc.subcore_barrier()

    lo = cid * NHALF
    base0 = sid * EPT1

    def chunk(c, carry):
        base = base0 + c * C1
        pltpu.sync_copy(idx_hbm.at[pl.ds(base, C1)], ibuf)
        pltpu.sync_copy(edge_hbm.at[pl.ds(base, C1)],
                        ebuf.at[pl.ds(0, C1), pl.ds(0, D)])

        def grp(j, cc):
            iv = ibuf[pl.ds(j * 16, 16)]
            local = iv - lo
            ok = (local >= 0) & (local < NHALF)
            wbuf[pl.ds(j * 16, 16)] = jnp.where(ok, local, NHALF + lanes)
            return cc

        lax.fori_loop(0, C1 // 16, grp, 0)
        pltpu.sync_copy(ebuf, acc.at[wbuf], add=True)
        return carry

    lax.fori_loop(0, NCHUNK1, chunk, 0)
    plsc.subcore_barrier()
    pltpu.sync_copy(acc.at[pl.ds(r0, RPT1)],
                    out_hbm.at[cid, pl.ds(r0, RPT1)])


_sums_call = pl.kernel(
    _sums_body,
    out_type=jax.ShapeDtypeStruct((NC, ACCR, DW), jnp.float32),
    mesh=_mesh,
    scratch_types=(
        pltpu.VMEM((C1, DW), jnp.float32),
        pltpu.VMEM((C1,), jnp.int32),
        pltpu.VMEM((C1,), jnp.int32),
        pltpu.VMEM_SHARED((ACCR, DW), jnp.float32),
    ),
    compiler_params=_sc_params,
)


BN = 256


def _combine_body(s_ref, na_ref, w0a_ref, w0b_ref, b0_ref,
                  wuv_ref, buv_ref, h_ref, uv_ref):
    s = s_ref[...]
    cnt = s[:, D:D + 1]
    mean = s[:, 0:D] / jnp.maximum(cnt, 1.0)
    acc = jnp.dot(na_ref[...], w0a_ref[...], preferred_element_type=jnp.float32)
    acc = acc + jnp.dot(mean, w0b_ref[...], preferred_element_type=jnp.float32)
    h = jax.nn.sigmoid(acc + b0_ref[...])
    h_ref[...] = h
    uv_ref[...] = jnp.dot(h, wuv_ref[...], preferred_element_type=jnp.float32) + buv_ref[...]


_combine_call = pl.pallas_call(
    _combine_body,
    grid=(NPAD // BN,),
    in_specs=[
        pl.BlockSpec((BN, DW), lambda i: (i, 0)),
        pl.BlockSpec((BN, D), lambda i: (i, 0)),
        pl.BlockSpec((D, D), lambda i: (0, 0)),
        pl.BlockSpec((D, D), lambda i: (0, 0)),
        pl.BlockSpec((1, D), lambda i: (0, 0)),
        pl.BlockSpec((D, 16), lambda i: (0, 0)),
        pl.BlockSpec((1, 16), lambda i: (0, 0)),
    ],
    out_specs=[
        pl.BlockSpec((BN, D), lambda i: (i, 0)),
        pl.BlockSpec((BN, 16), lambda i: (i, 0)),
    ],
    out_shape=[
        jax.ShapeDtypeStruct((NPAD, D), jnp.float32),
        jax.ShapeDtypeStruct((NPAD, 16), jnp.float32),
    ],
)


def _emb_body(h_hbm, src_hbm, dst_hbm, emb_hbm, sbuf, dbuf, rowss, rowsd, sem):
    cid = lax.axis_index("c")
    sid = lax.axis_index("s")
    wid = cid * NS + sid
    base0 = wid * EPW

    def chunk(c, carry):
        base = base0 + c * C3
        pltpu.sync_copy(src_hbm.at[pl.ds(base, C3)], sbuf)
        pltpu.sync_copy(dst_hbm.at[pl.ds(base, C3)], dbuf)
        cps = pltpu.async_copy(h_hbm.at[sbuf], rowss, sem)
        cpd = pltpu.async_copy(h_hbm.at[dbuf], rowsd, sem)
        cps.wait()
        cpd.wait()
        pltpu.sync_copy(rowss, emb_hbm.at[pl.ds(base, C3), pl.ds(0, D)])
        pltpu.sync_copy(rowsd, emb_hbm.at[pl.ds(base, C3), pl.ds(D, D)])
        return carry

    lax.fori_loop(0, NCHUNK3, chunk, 0)


_emb_call = pl.kernel(
    _emb_body,
    out_type=jax.ShapeDtypeStruct((N_EDGES, 2 * D), jnp.float32),
    mesh=_mesh,
    scratch_types=(
        pltpu.VMEM((C3,), jnp.int32),
        pltpu.VMEM((C3,), jnp.int32),
        pltpu.VMEM((C3, D), jnp.float32),
        pltpu.VMEM((C3, D), jnp.float32),
        pltpu.SemaphoreType.DMA,
    ),
)


def _probs_body(uv_hbm, ii_hbm, probs_hbm, iibuf, uvrows, pbuf, sem):
    cid = lax.axis_index("c")
    sid = lax.axis_index("s")
    wid = cid * NS + sid
    base0 = wid * EPW
    lanes = lax.iota(jnp.int32, 16)
    cu = jnp.zeros((16,), jnp.int32)
    cv = jnp.ones((16,), jnp.int32)

    def chunk(c, carry):
        base = base0 + c * CP
        pltpu.sync_copy(ii_hbm.at[pl.ds(2 * base, 2 * CP)], iibuf)
        pltpu.async_copy(uv_hbm.at[iibuf], uvrows, sem).wait()

        def grp(j, cc):
            s_idx = 32 * j + 2 * lanes
            u = plsc.load_gather(uvrows, [s_idx, cu])
            v = plsc.load_gather(uvrows, [s_idx + 1, cv])
            x = u + v
            pbuf[pl.ds(j * 16, 16)] = 1.0 / (1.0 + jnp.exp(-x))
            return cc

        lax.fori_loop(0, CP // 16, grp, 0)
        pltpu.sync_copy(pbuf, probs_hbm.at[pl.ds(base, CP)])
        return carry

    lax.fori_loop(0, NCHUNKP, chunk, 0)


_probs_call = pl.kernel(
    _probs_body,
    out_type=jax.ShapeDtypeStruct((N_EDGES,), jnp.float32),
    mesh=_mesh,
    scratch_types=(
        pltpu.VMEM((2 * CP,), jnp.int32),
        pltpu.VMEM((2 * CP, 16), jnp.float32),
        pltpu.VMEM((CP,), jnp.float32),
        pltpu.SemaphoreType.DMA,
    ),
    compiler_params=_sc_params,
)


def kernel(edge_attr, edge_index, node_attr, W0, b0, W_out, b_out):
    src = edge_index[0].astype(jnp.int32)
    dst = edge_index[1].astype(jnp.int32)
    ii = jnp.stack([src, dst], axis=1).reshape(-1)

    z = jnp.zeros((ACCR, DW), jnp.float32)
    onescol = jnp.zeros((C1, 16), jnp.float32).at[:, 0].set(1.0)
    ps = _sums_call(edge_attr, src, z, onescol)
    sums2 = jnp.concatenate([ps[0, :NHALF], ps[1, :NHALF]], axis=0)

    na_pad = jnp.pad(node_attr, ((0, NPAD - N_NODES), (0, 0)))
    w0a = W0[:, :D].T
    w0b = W0[:, D:].T
    wu = W_out[0, :D]
    wv = W_out[0, D:]
    wuv = jnp.concatenate(
        [jnp.stack([wu, wv], axis=1), jnp.zeros((D, 14), jnp.float32)], axis=1
    )
    buv = jnp.zeros((1, 16), jnp.float32).at[0, 0].set(b_out[0])
    h, uvt = _combine_call(sums2, na_pad, w0a, w0b, b0.reshape(1, D), wuv, buv)

    emb = _emb_call(h, src, dst)
    probs_flat = _probs_call(uvt, ii)
    return probs_flat.reshape(N_EDGES, 1), emb
